# baseline (speedup 1.0000x reference)
import functools

import jax
import jax.numpy as jnp
from jax import lax
from jax.experimental import pallas as pl
from jax.experimental.pallas import tpu as pltpu
from jax.experimental.pallas import tpu_sc as plsc

_F32 = jnp.float32


def _sc_gather_rows(table, idx):
    V, D = table.shape
    B = idx.shape[0]
    info = plsc.get_sparse_core_info()
    NC, NS = info.num_cores, info.num_subcores
    NW = NC * NS
    b_per_w = B // NW
    mesh = plsc.VectorSubcoreMesh(core_axis_name="c", subcore_axis_name="s")

    @functools.partial(
        pl.kernel, mesh=mesh,
        out_type=jax.ShapeDtypeStruct((B, D), _F32),
        scratch_types=[
            pltpu.VMEM((b_per_w,), jnp.int32),
            pltpu.VMEM((b_per_w, D), _F32),
            pltpu.SemaphoreType.DMA,
        ],
    )
    def k(table_hbm, idx_hbm, out_hbm, idx_v, rows_v, sem):
        wid = lax.axis_index("s") * NC + lax.axis_index("c")
        base = wid * b_per_w
        pltpu.sync_copy(idx_hbm.at[pl.ds(base, b_per_w)], idx_v)
        pltpu.async_copy(table_hbm.at[idx_v], rows_v, sem).wait()
        pltpu.sync_copy(rows_v, out_hbm.at[pl.ds(base, b_per_w)])

    return k(table, idx)


def _proj(x, ws, block_rows=256):
    N, di = x.shape
    K = len(ws)

    def body(*refs):
        x_ref = refs[0]
        xb = x_ref[...]
        for w_ref, o_ref in zip(refs[1:1 + K], refs[1 + K:]):
            o_ref[...] = jnp.dot(xb, w_ref[...], preferred_element_type=_F32)

    in_specs = [pl.BlockSpec((block_rows, di), lambda i: (i, 0))]
    in_specs += [pl.BlockSpec(w.shape, lambda i: (0, 0)) for w in ws]
    out_specs = [pl.BlockSpec((block_rows, w.shape[1]), lambda i: (i, 0))
                 for w in ws]
    out_shape = [jax.ShapeDtypeStruct((N, w.shape[1]), _F32) for w in ws]
    outs = pl.pallas_call(
        body,
        grid=(N // block_rows,),
        in_specs=in_specs,
        out_specs=out_specs,
        out_shape=out_shape,
    )(x, *ws)
    return list(outs)


def _attn(L, h, a1, a2, bd, pd, bu, pu, block_rows=256):
    N, do = h.shape
    has_d = bd is not None
    has_u = bu is not None

    def body(*refs):
        it = iter(refs)
        L_ref, h_ref, a1_ref, a2_ref = next(it), next(it), next(it), next(it)
        bd_ref = next(it) if has_d else None
        pd_ref = next(it) if has_d else None
        bu_ref = next(it) if has_u else None
        pu_ref = next(it) if has_u else None
        o_ref = next(it)

        i = pl.program_id(0)
        hf = h_ref[...]
        hb = h_ref[pl.ds(i * block_rows, block_rows), :]
        s1 = lax.dot_general(hb, a1_ref[...], (((1,), (1,)), ((), ())),
                             preferred_element_type=_F32)
        s2 = lax.dot_general(a2_ref[...], hf, (((1,), (1,)), ((), ())),
                             preferred_element_type=_F32)
        e = s1 + s2
        e = jnp.where(e >= 0, e, 0.2 * e)
        e = jnp.where(L_ref[...] != 0, e, -1e9)
        m = jnp.max(e, axis=1, keepdims=True)
        w = jnp.exp(e - m)
        den = jnp.sum(w, axis=1, keepdims=True)
        h16 = hf.astype(jnp.bfloat16)
        acc = jnp.dot(w.astype(jnp.bfloat16), h16,
                      preferred_element_type=_F32) / den
        if has_d:
            acc += lax.dot_general(bd_ref[...],
                                   pd_ref[...].astype(jnp.bfloat16),
                                   (((0,), (0,)), ((), ())),
                                   preferred_element_type=_F32)
        if has_u:
            acc += jnp.dot(bu_ref[...],
                           pu_ref[...].astype(jnp.bfloat16),
                           preferred_element_type=_F32)
        o_ref[...] = jnp.maximum(acc, 0.0)

    in_specs = [
        pl.BlockSpec((block_rows, N), lambda i: (i, 0)),
        pl.BlockSpec((N, do), lambda i: (0, 0)),
        pl.BlockSpec((1, do), lambda i: (0, 0)),
        pl.BlockSpec((1, do), lambda i: (0, 0)),
    ]
    args = [L, h, a1, a2]
    if has_d:
        np_ = bd.shape[0]
        in_specs += [pl.BlockSpec((np_, block_rows), lambda i: (0, i)),
                     pl.BlockSpec((np_, do), lambda i: (0, 0))]
        args += [bd, pd]
    if has_u:
        nn_ = bu.shape[1]
        in_specs += [pl.BlockSpec((block_rows, nn_), lambda i: (i, 0)),
                     pl.BlockSpec((nn_, do), lambda i: (0, 0))]
        args += [bu, pu]

    return pl.pallas_call(
        body,
        grid=(N // block_rows,),
        in_specs=in_specs,
        out_specs=pl.BlockSpec((block_rows, do), lambda i: (i, 0)),
        out_shape=jax.ShapeDtypeStruct((N, do), _F32),
    )(*args)


def _final(Lg, hg, h0, a1, a2, Bg, pu, wrel, brel):
    B = Lg.shape[0]
    N, do = h0.shape
    C = wrel.shape[1]

    def body(Lg_ref, hg_ref, h0_ref, a1_ref, a2_ref, Bg_ref, pu_ref,
             wrel_ref, brel_ref, o_ref):
        hf = h0_ref[...]
        s1 = lax.dot_general(hg_ref[...], a1_ref[...], (((1,), (1,)), ((), ())),
                             preferred_element_type=_F32)
        s2 = lax.dot_general(a2_ref[...], hf, (((1,), (1,)), ((), ())),
                             preferred_element_type=_F32)
        e = s1 + s2
        e = jnp.where(e >= 0, e, 0.2 * e)
        e = jnp.where(Lg_ref[...] != 0, e, -1e9)
        m = jnp.max(e, axis=1, keepdims=True)
        w = jnp.exp(e - m)
        den = jnp.sum(w, axis=1, keepdims=True)
        acc = jnp.dot(w.astype(jnp.bfloat16), hf.astype(jnp.bfloat16),
                      preferred_element_type=_F32) / den
        acc += jnp.dot(Bg_ref[...].astype(jnp.bfloat16),
                       pu_ref[...].astype(jnp.bfloat16),
                       preferred_element_type=_F32)
        acc = jnp.maximum(acc, 0.0)
        o_ref[...] = (jnp.dot(acc, wrel_ref[...], preferred_element_type=_F32)
                      + brel_ref[...])

    return pl.pallas_call(
        body,
        out_shape=jax.ShapeDtypeStruct((B, C), _F32),
    )(Lg, hg, h0, a1, a2, Bg, pu, wrel, brel)


def _split_a(lp):
    a = lp["a"]
    do = a.shape[0] // 2
    return a[:do].reshape(1, do), a[do:].reshape(1, do)


def kernel(emb0, emb1, emb2, emb3, lap0, lap1, lap2, lap3,
           bnd1, bnd2, bnd3, order, idx, rel, params):
    del order
    idx = idx.astype(jnp.int32)
    laps = [x.astype(jnp.bfloat16) for x in (lap0, lap1, lap2, lap3)]
    bnds = [None] + [x.astype(jnp.bfloat16) for x in (bnd1, bnd2, bnd3)]

    Lg = _sc_gather_rows(lap0, idx)
    Bg = _sc_gather_rows(bnd1, idx)

    xs = [emb0, emb1, emb2, emb3]

    need = {
        1: {0: "h d", 1: "h d u", 2: "h d u", 3: "h u"},
        2: {0: "h d", 1: "h d u", 2: "h u", 3: "u"},
        3: {0: "h d", 1: "h u", 2: "u"},
    }
    orders_per_layer = {1: (0, 1, 2, 3), 2: (0, 1, 2), 3: (0, 1)}

    for lnum in (1, 2, 3):
        lp = params["l%d" % lnum]
        a1, a2 = _split_a(lp)
        h, hd, hu = {}, {}, {}
        for i, spec in need[lnum].items():
            toks = spec.split()
            ws, dsts = [], []
            if "h" in toks:
                ws.append(lp["W"]); dsts.append((h, i))
            if "d" in toks:
                ws.append(lp["Wd"]); dsts.append((hd, i))
            if "u" in toks:
                ws.append(lp["Wu"]); dsts.append((hu, i))
            outs = _proj(xs[i], ws)
            for (dct, key), o in zip(dsts, outs):
                dct[key] = o
        nxt = [None, None, None, None]
        for i in orders_per_layer[lnum]:
            bd = bnds[i] if i > 0 else None
            pd = hd.get(i - 1) if i > 0 else None
            bu = bnds[i + 1] if i < 3 else None
            pu = hu.get(i + 1) if i < 3 else None
            nxt[i] = _attn(laps[i], h[i], a1, a2, bd, pd, bu, pu)
        xs = nxt

    lp = params["l4"]
    a1, a2 = _split_a(lp)
    (h0,) = _proj(xs[0], [lp["W"]])
    (pu1,) = _proj(xs[1], [lp["Wu"]])
    hg = _sc_gather_rows(h0, idx)

    rows = _final(Lg, hg, h0, a1, a2, Bg, pu1,
                  params["W_rel"], params["b_rel"].reshape(1, -1))

    nz = jnp.stack(jnp.nonzero(rel, size=rel.shape[0]), axis=1)
    return rows[nz]


# speedup vs baseline: 1.0128x; 1.0128x over previous
import functools

import jax
import jax.numpy as jnp
from jax import lax
from jax.experimental import pallas as pl
from jax.experimental.pallas import tpu as pltpu
from jax.experimental.pallas import tpu_sc as plsc

_F32 = jnp.float32
_BF16 = jnp.bfloat16


def _sc_gather_rows(table, idx):
    V, D = table.shape
    B = idx.shape[0]
    info = plsc.get_sparse_core_info()
    NC, NS = info.num_cores, info.num_subcores
    NW = NC * NS
    b_per_w = B // NW
    mesh = plsc.VectorSubcoreMesh(core_axis_name="c", subcore_axis_name="s")

    @functools.partial(
        pl.kernel, mesh=mesh,
        out_type=jax.ShapeDtypeStruct((B, D), table.dtype),
        scratch_types=[
            pltpu.VMEM((b_per_w,), jnp.int32),
            pltpu.VMEM((b_per_w, D), table.dtype),
            pltpu.SemaphoreType.DMA,
        ],
    )
    def k(table_hbm, idx_hbm, out_hbm, idx_v, rows_v, sem):
        wid = lax.axis_index("s") * NC + lax.axis_index("c")
        base = wid * b_per_w
        pltpu.sync_copy(idx_hbm.at[pl.ds(base, b_per_w)], idx_v)
        pltpu.async_copy(table_hbm.at[idx_v], rows_v, sem).wait()
        pltpu.sync_copy(rows_v, out_hbm.at[pl.ds(base, b_per_w)])

    return k(table, idx)


def _nz_mask(L, block_rows=512):
    N = L.shape[0]

    def body(L_ref, m_ref):
        m_ref[...] = (L_ref[...] != 0).astype(jnp.int8)

    return pl.pallas_call(
        body,
        grid=(N // block_rows,),
        in_specs=[pl.BlockSpec((block_rows, N), lambda i: (i, 0))],
        out_specs=pl.BlockSpec((block_rows, N), lambda i: (i, 0)),
        out_shape=jax.ShapeDtypeStruct((N, N), jnp.int8),
    )(L)


def _proj(x, ws, a1=None, a2=None, out_dtype=_BF16, block_rows=256):
    N, di = x.shape
    K = len(ws)
    with_scores = a1 is not None

    def body(*refs):
        it = iter(refs)
        x_ref = next(it)
        w_refs = [next(it) for _ in range(K)]
        if with_scores:
            a1_ref, a2_ref = next(it), next(it)
        o_refs = [next(it) for _ in range(K)]
        xb = x_ref[...]
        for k, (w_ref, o_ref) in enumerate(zip(w_refs, o_refs)):
            hf = jnp.dot(xb, w_ref[...], preferred_element_type=_F32)
            o_ref[...] = hf.astype(out_dtype)
            if with_scores and k == 0:
                s1_ref, s2_ref = next(it), next(it)
                s1_ref[...] = lax.dot_general(
                    hf, a1_ref[...], (((1,), (1,)), ((), ())),
                    preferred_element_type=_F32)
                s2_ref[...] = lax.dot_general(
                    a2_ref[...], hf, (((1,), (1,)), ((), ())),
                    preferred_element_type=_F32)

    in_specs = [pl.BlockSpec((block_rows, di), lambda i: (i, 0))]
    in_specs += [pl.BlockSpec(w.shape, lambda i: (0, 0)) for w in ws]
    args = [x] + list(ws)
    out_specs = [pl.BlockSpec((block_rows, w.shape[1]), lambda i: (i, 0))
                 for w in ws]
    out_shape = [jax.ShapeDtypeStruct((N, w.shape[1]), out_dtype) for w in ws]
    if with_scores:
        in_specs += [pl.BlockSpec(a1.shape, lambda i: (0, 0)),
                     pl.BlockSpec(a2.shape, lambda i: (0, 0))]
        args += [a1, a2]
        out_specs += [pl.BlockSpec((block_rows, 1), lambda i: (i, 0)),
                      pl.BlockSpec((1, block_rows), lambda i: (0, i))]
        out_shape += [jax.ShapeDtypeStruct((N, 1), _F32),
                      jax.ShapeDtypeStruct((1, N), _F32)]
    outs = pl.pallas_call(
        body,
        grid=(N // block_rows,),
        in_specs=in_specs,
        out_specs=out_specs,
        out_shape=out_shape,
    )(*args)
    return list(outs)


def _attn(M, h, s1, s2, bd, pd, bu, pu, block_rows=256):
    N, do = h.shape
    has_d = bd is not None
    has_u = bu is not None

    def body(*refs):
        it = iter(refs)
        M_ref, h_ref, s1_ref, s2_ref = next(it), next(it), next(it), next(it)
        bd_ref = next(it) if has_d else None
        pd_ref = next(it) if has_d else None
        bu_ref = next(it) if has_u else None
        pu_ref = next(it) if has_u else None
        o_ref = next(it)

        e = s1_ref[...] + s2_ref[...]
        e = jnp.where(e >= 0, e, 0.2 * e)
        e = jnp.where(M_ref[...] != 0, e, -1e9)
        m = jnp.max(e, axis=1, keepdims=True)
        w = jnp.exp(e - m)
        den = jnp.sum(w, axis=1, keepdims=True)
        acc = jnp.dot(w.astype(_BF16), h_ref[...],
                      preferred_element_type=_F32) / den
        if has_d:
            acc += lax.dot_general(bd_ref[...], pd_ref[...],
                                   (((0,), (0,)), ((), ())),
                                   preferred_element_type=_F32)
        if has_u:
            acc += jnp.dot(bu_ref[...], pu_ref[...],
                           preferred_element_type=_F32)
        o_ref[...] = jnp.maximum(acc, 0.0).astype(_BF16)

    in_specs = [
        pl.BlockSpec((block_rows, N), lambda i: (i, 0)),
        pl.BlockSpec((N, do), lambda i: (0, 0)),
        pl.BlockSpec((block_rows, 1), lambda i: (i, 0)),
        pl.BlockSpec((1, N), lambda i: (0, 0)),
    ]
    args = [M, h, s1, s2]
    if has_d:
        np_ = bd.shape[0]
        in_specs += [pl.BlockSpec((np_, block_rows), lambda i: (0, i)),
                     pl.BlockSpec((np_, do), lambda i: (0, 0))]
        args += [bd, pd]
    if has_u:
        nn_ = bu.shape[1]
        in_specs += [pl.BlockSpec((block_rows, nn_), lambda i: (i, 0)),
                     pl.BlockSpec((nn_, do), lambda i: (0, 0))]
        args += [bu, pu]

    return pl.pallas_call(
        body,
        grid=(N // block_rows,),
        in_specs=in_specs,
        out_specs=pl.BlockSpec((block_rows, do), lambda i: (i, 0)),
        out_shape=jax.ShapeDtypeStruct((N, do), _BF16),
    )(*args)


def _final(Lg, hg, h0, a1, a2, Bg, pu, wrel, brel):
    B = Lg.shape[0]
    N, do = h0.shape
    C = wrel.shape[1]

    def body(Lg_ref, hg_ref, h0_ref, a1_ref, a2_ref, Bg_ref, pu_ref,
             wrel_ref, brel_ref, o_ref):
        hf = h0_ref[...]
        s1 = lax.dot_general(hg_ref[...], a1_ref[...], (((1,), (1,)), ((), ())),
                             preferred_element_type=_F32)
        s2 = lax.dot_general(a2_ref[...], hf, (((1,), (1,)), ((), ())),
                             preferred_element_type=_F32)
        e = s1 + s2
        e = jnp.where(e >= 0, e, 0.2 * e)
        e = jnp.where(Lg_ref[...] != 0, e, -1e9)
        m = jnp.max(e, axis=1, keepdims=True)
        w = jnp.exp(e - m)
        den = jnp.sum(w, axis=1, keepdims=True)
        acc = jnp.dot(w.astype(_BF16), hf.astype(_BF16),
                      preferred_element_type=_F32) / den
        acc += jnp.dot(Bg_ref[...].astype(_BF16), pu_ref[...],
                       preferred_element_type=_F32)
        acc = jnp.maximum(acc, 0.0)
        o_ref[...] = (jnp.dot(acc, wrel_ref[...], preferred_element_type=_F32)
                      + brel_ref[...])

    return pl.pallas_call(
        body,
        out_shape=jax.ShapeDtypeStruct((B, C), _F32),
    )(Lg, hg, h0, a1, a2, Bg, pu, wrel, brel)


def _split_a(lp):
    a = lp["a"]
    do = a.shape[0] // 2
    return a[:do].reshape(1, do), a[do:].reshape(1, do)


def kernel(emb0, emb1, emb2, emb3, lap0, lap1, lap2, lap3,
           bnd1, bnd2, bnd3, order, idx, rel, params):
    del order
    idx = idx.astype(jnp.int32)

    Lg = _sc_gather_rows(lap0, idx)
    Bg = _sc_gather_rows(bnd1, idx)

    masks = [_nz_mask(lap0), _nz_mask(lap1), _nz_mask(lap2), _nz_mask(lap3)]
    bnds = [None] + [x.astype(_BF16) for x in (bnd1, bnd2, bnd3)]

    xs = [x.astype(_BF16) for x in (emb0, emb1, emb2, emb3)]

    need = {
        1: {0: "h d", 1: "h d u", 2: "h d u", 3: "h u"},
        2: {0: "h d", 1: "h d u", 2: "h u", 3: "u"},
        3: {0: "h d", 1: "h u", 2: "u"},
    }
    orders_per_layer = {1: (0, 1, 2, 3), 2: (0, 1, 2), 3: (0, 1)}

    for lnum in (1, 2, 3):
        lp = params["l%d" % lnum]
        a1, a2 = _split_a(lp)
        h, hd, hu, s1d, s2d = {}, {}, {}, {}, {}
        for i, spec in need[lnum].items():
            toks = spec.split()
            ws, dsts = [], []
            if "h" in toks:
                ws.append(lp["W"].astype(_BF16)); dsts.append((h, i))
            if "d" in toks:
                ws.append(lp["Wd"].astype(_BF16)); dsts.append((hd, i))
            if "u" in toks:
                ws.append(lp["Wu"].astype(_BF16)); dsts.append((hu, i))
            if "h" in toks:
                outs = _proj(xs[i], ws, a1, a2)
                s1d[i], s2d[i] = outs[-2], outs[-1]
                outs = outs[:-2]
            else:
                outs = _proj(xs[i], ws)
            for (dct, key), o in zip(dsts, outs):
                dct[key] = o
        nxt = [None, None, None, None]
        for i in orders_per_layer[lnum]:
            bd = bnds[i] if i > 0 else None
            pd = hd.get(i - 1) if i > 0 else None
            bu = bnds[i + 1] if i < 3 else None
            pu = hu.get(i + 1) if i < 3 else None
            nxt[i] = _attn(masks[i], h[i], s1d[i], s2d[i], bd, pd, bu, pu)
        xs = nxt

    lp = params["l4"]
    a1, a2 = _split_a(lp)
    (h0,) = _proj(xs[0], [lp["W"].astype(_BF16)], out_dtype=_F32)
    (pu1,) = _proj(xs[1], [lp["Wu"].astype(_BF16)])
    hg = _sc_gather_rows(h0, idx)

    rows = _final(Lg, hg, h0, a1, a2, Bg, pu1,
                  params["W_rel"], params["b_rel"].reshape(1, -1))

    nz = jnp.stack(jnp.nonzero(rel, size=rel.shape[0]), axis=1)
    return rows[nz]


# speedup vs baseline: 1.0873x; 1.0736x over previous
import functools

import jax
import jax.numpy as jnp
from jax import lax
from jax.experimental import pallas as pl
from jax.experimental.pallas import tpu as pltpu
from jax.experimental.pallas import tpu_sc as plsc

_F32 = jnp.float32


def _sc_gather_rows(table, idx):
    V, D = table.shape
    B = idx.shape[0]
    info = plsc.get_sparse_core_info()
    NC, NS = info.num_cores, info.num_subcores
    NW = NC * NS
    b_per_w = B // NW
    mesh = plsc.VectorSubcoreMesh(core_axis_name="c", subcore_axis_name="s")

    @functools.partial(
        pl.kernel, mesh=mesh,
        out_type=jax.ShapeDtypeStruct((B, D), table.dtype),
        scratch_types=[
            pltpu.VMEM((b_per_w,), jnp.int32),
            pltpu.VMEM((b_per_w, D), table.dtype),
            pltpu.SemaphoreType.DMA,
        ],
    )
    def k(table_hbm, idx_hbm, out_hbm, idx_v, rows_v, sem):
        wid = lax.axis_index("s") * NC + lax.axis_index("c")
        base = wid * b_per_w
        pltpu.sync_copy(idx_hbm.at[pl.ds(base, b_per_w)], idx_v)
        pltpu.async_copy(table_hbm.at[idx_v], rows_v, sem).wait()
        pltpu.sync_copy(rows_v, out_hbm.at[pl.ds(base, b_per_w)])

    return k(table, idx)


_BR = 256


def _proj(x, ws, a1=None, a2=None, block_rows=_BR):
    N, di = x.shape
    K = len(ws)
    with_scores = a1 is not None

    def body(*refs):
        it = iter(refs)
        x_ref = next(it)
        w_refs = [next(it) for _ in range(K)]
        if with_scores:
            a1_ref, a2_ref = next(it), next(it)
        o_refs = [next(it) for _ in range(K)]
        xb = x_ref[...]
        for k, (w_ref, o_ref) in enumerate(zip(w_refs, o_refs)):
            hf = jnp.dot(xb, w_ref[...], preferred_element_type=_F32)
            o_ref[...] = hf
            if with_scores and k == 0:
                s1_ref, s2_ref = next(it), next(it)
                s1_ref[...] = lax.dot_general(
                    hf, a1_ref[...], (((1,), (1,)), ((), ())),
                    preferred_element_type=_F32)
                s2_ref[...] = lax.dot_general(
                    a2_ref[...], hf, (((1,), (1,)), ((), ())),
                    preferred_element_type=_F32)

    in_specs = [pl.BlockSpec((block_rows, di), lambda i: (i, 0))]
    in_specs += [pl.BlockSpec(w.shape, lambda i: (0, 0)) for w in ws]
    args = [x] + list(ws)
    out_specs = [pl.BlockSpec((block_rows, w.shape[1]), lambda i: (i, 0))
                 for w in ws]
    out_shape = [jax.ShapeDtypeStruct((N, w.shape[1]), _F32) for w in ws]
    if with_scores:
        in_specs += [pl.BlockSpec(a1.shape, lambda i: (0, 0)),
                     pl.BlockSpec(a2.shape, lambda i: (0, 0))]
        args += [a1, a2]
        out_specs += [pl.BlockSpec((block_rows, 1), lambda i: (i, 0)),
                      pl.BlockSpec((1, block_rows), lambda i: (0, i))]
        out_shape += [jax.ShapeDtypeStruct((N, 1), _F32),
                      jax.ShapeDtypeStruct((1, N), _F32)]
    outs = pl.pallas_call(
        body,
        grid=(N // block_rows,),
        in_specs=in_specs,
        out_specs=out_specs,
        out_shape=out_shape,
    )(*args)
    return list(outs)


def _attn(L, ha, s1a, s2a, off, bd, pda, doff, bu, pua, uoff,
          block_rows=_BR):
    N = L.shape[0]
    do = ha.shape[1]
    has_d = bd is not None
    has_u = bu is not None

    def body(*refs):
        it = iter(refs)
        L_ref, h_ref, s1_ref, s2_ref = next(it), next(it), next(it), next(it)
        bd_ref = next(it) if has_d else None
        pd_ref = next(it) if has_d else None
        bu_ref = next(it) if has_u else None
        pu_ref = next(it) if has_u else None
        o_ref = next(it)

        e = s1_ref[...] + s2_ref[...]
        e = jnp.where(e >= 0, e, 0.2 * e)
        e = jnp.where(L_ref[...] != 0, e, -1e9)
        m = jnp.max(e, axis=1, keepdims=True)
        w = jnp.exp(e - m)
        den = jnp.sum(w, axis=1, keepdims=True)
        acc = jnp.dot(w, h_ref[...], preferred_element_type=_F32) / den
        if has_d:
            acc += lax.dot_general(bd_ref[...], pd_ref[...],
                                   (((0,), (0,)), ((), ())),
                                   preferred_element_type=_F32)
        if has_u:
            acc += jnp.dot(bu_ref[...], pu_ref[...],
                           preferred_element_type=_F32)
        o_ref[...] = jnp.maximum(acc, 0.0)

    hb = off // N
    sb = off // block_rows
    in_specs = [
        pl.BlockSpec((block_rows, N), lambda i: (i, 0)),
        pl.BlockSpec((N, do), lambda i, b=hb: (b, 0)),
        pl.BlockSpec((block_rows, 1), lambda i, b=sb: (b + i, 0)),
        pl.BlockSpec((1, N), lambda i, b=hb: (0, b)),
    ]
    args = [L, ha, s1a, s2a]
    if has_d:
        npv = bd.shape[0]
        db = doff // npv
        in_specs += [pl.BlockSpec((npv, block_rows), lambda i: (0, i)),
                     pl.BlockSpec((npv, do), lambda i, b=db: (b, 0))]
        args += [bd, pda]
    if has_u:
        nnv = bu.shape[1]
        ub = uoff // nnv
        in_specs += [pl.BlockSpec((block_rows, nnv), lambda i: (i, 0)),
                     pl.BlockSpec((nnv, do), lambda i, b=ub: (b, 0))]
        args += [bu, pua]

    return pl.pallas_call(
        body,
        grid=(N // block_rows,),
        in_specs=in_specs,
        out_specs=pl.BlockSpec((block_rows, do), lambda i: (i, 0)),
        out_shape=jax.ShapeDtypeStruct((N, do), _F32),
    )(*args)


def _final(Lg, idxc, s1, s2, h0, Bg, pu, wrel, brel):
    B = Lg.shape[0]
    N, do = h0.shape
    C = wrel.shape[1]

    def body(Lg_ref, idx_ref, s1_ref, s2_ref, h0_ref, Bg_ref, pu_ref,
             wrel_ref, brel_ref, o_ref):
        cols = lax.broadcasted_iota(jnp.int32, (B, N), 1)
        oh = (cols == idx_ref[...]).astype(_F32)
        s1g = jnp.dot(oh, s1_ref[...], preferred_element_type=_F32)
        e = s1g + s2_ref[...]
        e = jnp.where(e >= 0, e, 0.2 * e)
        e = jnp.where(Lg_ref[...] != 0, e, -1e9)
        m = jnp.max(e, axis=1, keepdims=True)
        w = jnp.exp(e - m)
        den = jnp.sum(w, axis=1, keepdims=True)
        acc = jnp.dot(w, h0_ref[...], preferred_element_type=_F32) / den
        acc += jnp.dot(Bg_ref[...], pu_ref[...], preferred_element_type=_F32)
        acc = jnp.maximum(acc, 0.0)
        o_ref[...] = (jnp.dot(acc, wrel_ref[...], preferred_element_type=_F32)
                      + brel_ref[...])

    return pl.pallas_call(
        body,
        out_shape=jax.ShapeDtypeStruct((B, C), _F32),
    )(Lg, idxc, s1, s2, h0, Bg, pu, wrel, brel)


def _split_a(lp):
    a = lp["a"]
    do = a.shape[0] // 2
    return a[:do].reshape(1, do), a[do:].reshape(1, do)


def kernel(emb0, emb1, emb2, emb3, lap0, lap1, lap2, lap3,
           bnd1, bnd2, bnd3, order, idx, rel, params):
    del order
    idx = idx.astype(jnp.int32)

    Lg = _sc_gather_rows(lap0, idx)
    Bg = _sc_gather_rows(bnd1, idx)

    laps = [lap0, lap1, lap2, lap3]
    bnds = [None, bnd1, bnd2, bnd3]

    stack_order = {1: (0, 3, 1, 2), 2: (0, 3, 1, 2), 3: (1, 2, 0)}
    orders_per_layer = {1: (0, 1, 2, 3), 2: (0, 1, 2), 3: (0, 1)}

    xs = [emb0, emb1, emb2, emb3]
    for lnum in (1, 2, 3):
        lp = params["l%d" % lnum]
        a1, a2 = _split_a(lp)
        so = stack_order[lnum]
        offs = {}
        off = 0
        for i in so:
            offs[i] = off
            off += xs[i].shape[0]
        xcat = jnp.concatenate([xs[i] for i in so], axis=0)
        ha, hda, hua, s1a, s2a = _proj(
            xcat, [lp["W"], lp["Wd"], lp["Wu"]], a1, a2)
        nxt = [None, None, None, None]
        for i in orders_per_layer[lnum]:
            bd = bnds[i] if i > 0 else None
            doff = offs[i - 1] if i > 0 else 0
            bu = bnds[i + 1] if i < 3 else None
            uoff = offs[i + 1] if i < 3 else 0
            nxt[i] = _attn(laps[i], ha, s1a, s2a, offs[i],
                           bd, hda, doff, bu, hua, uoff)
        xs = nxt

    lp = params["l4"]
    a1, a2 = _split_a(lp)
    h0, s1, s2 = _proj(xs[0], [lp["W"]], a1, a2)
    (pu1,) = _proj(xs[1], [lp["Wu"]])

    rows = _final(Lg, idx.reshape(-1, 1), s1, s2, h0, Bg, pu1,
                  params["W_rel"], params["b_rel"].reshape(1, -1))

    nz = jnp.stack(jnp.nonzero(rel, size=rel.shape[0]), axis=1)
    return rows[nz]


# speedup vs baseline: 1.2967x; 1.1925x over previous
import functools

import jax
import jax.numpy as jnp
from jax import lax
from jax.experimental import pallas as pl
from jax.experimental.pallas import tpu as pltpu
from jax.experimental.pallas import tpu_sc as plsc

_F32 = jnp.float32
_BR = 256


def _sc_gather_rows(table, idx):
    V, D = table.shape
    B = idx.shape[0]
    info = plsc.get_sparse_core_info()
    NC, NS = info.num_cores, info.num_subcores
    NW = NC * NS
    b_per_w = B // NW
    mesh = plsc.VectorSubcoreMesh(core_axis_name="c", subcore_axis_name="s")

    @functools.partial(
        pl.kernel, mesh=mesh,
        out_type=jax.ShapeDtypeStruct((B, D), table.dtype),
        scratch_types=[
            pltpu.VMEM((b_per_w,), jnp.int32),
            pltpu.VMEM((b_per_w, D), table.dtype),
            pltpu.SemaphoreType.DMA,
        ],
    )
    def k(table_hbm, idx_hbm, out_hbm, idx_v, rows_v, sem):
        wid = lax.axis_index("s") * NC + lax.axis_index("c")
        base = wid * b_per_w
        pltpu.sync_copy(idx_hbm.at[pl.ds(base, b_per_w)], idx_v)
        pltpu.async_copy(table_hbm.at[idx_v], rows_v, sem).wait()
        pltpu.sync_copy(rows_v, out_hbm.at[pl.ds(base, b_per_w)])

    return k(table, idx)


def _proj(x, ws, a1, a2, block_rows=_BR):
    N, di = x.shape
    K = len(ws)

    def body(*refs):
        it = iter(refs)
        x_ref = next(it)
        w_refs = [next(it) for _ in range(K)]
        a1_ref, a2_ref = next(it), next(it)
        o_refs = [next(it) for _ in range(K)]
        s1_ref, s2_ref = next(it), next(it)
        xb = x_ref[...]
        for k, (w_ref, o_ref) in enumerate(zip(w_refs, o_refs)):
            hf = jnp.dot(xb, w_ref[...], preferred_element_type=_F32)
            o_ref[...] = hf
            if k == 0:
                s1_ref[...] = lax.dot_general(
                    hf, a1_ref[...], (((1,), (1,)), ((), ())),
                    preferred_element_type=_F32)
                s2_ref[...] = lax.dot_general(
                    a2_ref[...], hf, (((1,), (1,)), ((), ())),
                    preferred_element_type=_F32)

    in_specs = [pl.BlockSpec((block_rows, di), lambda i: (i, 0))]
    in_specs += [pl.BlockSpec(w.shape, lambda i: (0, 0)) for w in ws]
    in_specs += [pl.BlockSpec(a1.shape, lambda i: (0, 0)),
                 pl.BlockSpec(a2.shape, lambda i: (0, 0))]
    out_specs = [pl.BlockSpec((block_rows, w.shape[1]), lambda i: (i, 0))
                 for w in ws]
    out_specs += [pl.BlockSpec((block_rows, 1), lambda i: (i, 0)),
                  pl.BlockSpec((1, block_rows), lambda i: (0, i))]
    out_shape = [jax.ShapeDtypeStruct((N, w.shape[1]), _F32) for w in ws]
    out_shape += [jax.ShapeDtypeStruct((N, 1), _F32),
                  jax.ShapeDtypeStruct((1, N), _F32)]
    return list(pl.pallas_call(
        body,
        grid=(N // block_rows,),
        in_specs=in_specs,
        out_specs=out_specs,
        out_shape=out_shape,
    )(x, *ws, a1, a2))


def _attn(L, ha, s1a, s2a, off, bd, pda, doff, bu, pua, uoff,
          wnext, a1n=None, a2n=None, block_rows=_BR):
    N = L.shape[0]
    do = ha.shape[1]
    K = len(wnext)
    has_d = bd is not None
    has_u = bu is not None
    with_scores = a1n is not None

    def body(*refs):
        it = iter(refs)
        L_ref, h_ref, s1_ref, s2_ref = next(it), next(it), next(it), next(it)
        bd_ref = next(it) if has_d else None
        pd_ref = next(it) if has_d else None
        bu_ref = next(it) if has_u else None
        pu_ref = next(it) if has_u else None
        w_refs = [next(it) for _ in range(K)]
        if with_scores:
            a1_ref, a2_ref = next(it), next(it)
        o_refs = [next(it) for _ in range(K)]
        if with_scores:
            s1o_ref, s2o_ref = next(it), next(it)

        e = s1_ref[...] + s2_ref[...]
        e = jnp.where(e >= 0, e, 0.2 * e)
        e = jnp.where(L_ref[...] != 0, e, -1e9)
        m = jnp.max(e, axis=1, keepdims=True)
        w = jnp.exp(e - m)
        den = jnp.sum(w, axis=1, keepdims=True)
        acc = jnp.dot(w, h_ref[...], preferred_element_type=_F32) / den
        if has_d:
            acc += lax.dot_general(bd_ref[...], pd_ref[...],
                                   (((0,), (0,)), ((), ())),
                                   preferred_element_type=_F32)
        if has_u:
            acc += jnp.dot(bu_ref[...], pu_ref[...],
                           preferred_element_type=_F32)
        r = jnp.maximum(acc, 0.0)
        for k, (w_ref, o_ref) in enumerate(zip(w_refs, o_refs)):
            hf = jnp.dot(r, w_ref[...], preferred_element_type=_F32)
            o_ref[...] = hf
            if with_scores and k == 0:
                s1o_ref[...] = lax.dot_general(
                    hf, a1_ref[...], (((1,), (1,)), ((), ())),
                    preferred_element_type=_F32)
                s2o_ref[...] = lax.dot_general(
                    a2_ref[...], hf, (((1,), (1,)), ((), ())),
                    preferred_element_type=_F32)

    hb = off // N
    sb = off // block_rows
    in_specs = [
        pl.BlockSpec((block_rows, N), lambda i: (i, 0)),
        pl.BlockSpec((N, do), lambda i, b=hb: (b, 0)),
        pl.BlockSpec((block_rows, 1), lambda i, b=sb: (b + i, 0)),
        pl.BlockSpec((1, N), lambda i, b=hb: (0, b)),
    ]
    args = [L, ha, s1a, s2a]
    if has_d:
        npv = bd.shape[0]
        db = doff // npv
        in_specs += [pl.BlockSpec((npv, block_rows), lambda i: (0, i)),
                     pl.BlockSpec((npv, do), lambda i, b=db: (b, 0))]
        args += [bd, pda]
    if has_u:
        nnv = bu.shape[1]
        ub = uoff // nnv
        in_specs += [pl.BlockSpec((block_rows, nnv), lambda i: (i, 0)),
                     pl.BlockSpec((nnv, do), lambda i, b=ub: (b, 0))]
        args += [bu, pua]
    in_specs += [pl.BlockSpec(wk.shape, lambda i: (0, 0)) for wk in wnext]
    args += list(wnext)
    out_specs = [pl.BlockSpec((block_rows, wk.shape[1]), lambda i: (i, 0))
                 for wk in wnext]
    out_shape = [jax.ShapeDtypeStruct((N, wk.shape[1]), _F32)
                 for wk in wnext]
    if with_scores:
        in_specs += [pl.BlockSpec(a1n.shape, lambda i: (0, 0)),
                     pl.BlockSpec(a2n.shape, lambda i: (0, 0))]
        args += [a1n, a2n]
        out_specs += [pl.BlockSpec((block_rows, 1), lambda i: (i, 0)),
                      pl.BlockSpec((1, block_rows), lambda i: (0, i))]
        out_shape += [jax.ShapeDtypeStruct((N, 1), _F32),
                      jax.ShapeDtypeStruct((1, N), _F32)]

    return list(pl.pallas_call(
        body,
        grid=(N // block_rows,),
        in_specs=in_specs,
        out_specs=out_specs,
        out_shape=out_shape,
    )(*args))


def _final(Lg, idxc, s1, s2, h0, Bg, pu, wrel, brel):
    B = Lg.shape[0]
    N, do = h0.shape
    C = wrel.shape[1]

    def body(Lg_ref, idx_ref, s1_ref, s2_ref, h0_ref, Bg_ref, pu_ref,
             wrel_ref, brel_ref, o_ref):
        cols = lax.broadcasted_iota(jnp.int32, (B, N), 1)
        oh = (cols == idx_ref[...]).astype(_F32)
        s1g = jnp.dot(oh, s1_ref[...], preferred_element_type=_F32)
        e = s1g + s2_ref[...]
        e = jnp.where(e >= 0, e, 0.2 * e)
        e = jnp.where(Lg_ref[...] != 0, e, -1e9)
        m = jnp.max(e, axis=1, keepdims=True)
        w = jnp.exp(e - m)
        den = jnp.sum(w, axis=1, keepdims=True)
        acc = jnp.dot(w, h0_ref[...], preferred_element_type=_F32) / den
        acc += jnp.dot(Bg_ref[...], pu_ref[...], preferred_element_type=_F32)
        acc = jnp.maximum(acc, 0.0)
        o_ref[...] = (jnp.dot(acc, wrel_ref[...], preferred_element_type=_F32)
                      + brel_ref[...])

    return pl.pallas_call(
        body,
        out_shape=jax.ShapeDtypeStruct((B, C), _F32),
    )(Lg, idxc, s1, s2, h0, Bg, pu, wrel, brel)


def _split_a(lp):
    a = lp["a"]
    do = a.shape[0] // 2
    return a[:do].reshape(1, do), a[do:].reshape(1, do)


def kernel(emb0, emb1, emb2, emb3, lap0, lap1, lap2, lap3,
           bnd1, bnd2, bnd3, order, idx, rel, params):
    del order
    idx = idx.astype(jnp.int32)

    Lg = _sc_gather_rows(lap0, idx)
    Bg = _sc_gather_rows(bnd1, idx)

    laps = [lap0, lap1, lap2, lap3]
    bnds = [None, bnd1, bnd2, bnd3]
    l1, l2, l3, l4 = (params["l%d" % i] for i in (1, 2, 3, 4))
    a2p = _split_a(l2)
    a3p = _split_a(l3)
    a4p = _split_a(l4)
    wmap = {"h": "W", "d": "Wd", "u": "Wu"}

    so = (0, 3, 1, 2)
    embs = [emb0, emb1, emb2, emb3]
    offs = {}
    off = 0
    for i in so:
        offs[i] = off
        off += embs[i].shape[0]
    xcat = jnp.concatenate([embs[i] for i in so], axis=0)
    a1, a2 = _split_a(l1)
    ha, hda, hua, s1a, s2a = _proj(xcat, [l1["W"], l1["Wd"], l1["Wu"]], a1, a2)

    kinds = {0: "h d", 1: "h d u", 2: "h u", 3: "u"}
    h2, hd2, hu2, s12, s22 = {}, {}, {}, {}, {}
    for i in (0, 1, 2, 3):
        toks = kinds[i].split()
        ws = [l2[wmap[t]] for t in toks]
        sc = "h" in toks
        outs = _attn(laps[i], ha, s1a, s2a, offs[i],
                     bnds[i] if i > 0 else None, hda,
                     offs[i - 1] if i > 0 else 0,
                     bnds[i + 1] if i < 3 else None, hua,
                     offs[i + 1] if i < 3 else 0,
                     ws, a2p[0] if sc else None, a2p[1] if sc else None)
        dsts = {"h": h2, "d": hd2, "u": hu2}
        for t, o in zip(toks, outs[:len(toks)]):
            dsts[t][i] = o
        if sc:
            s12[i], s22[i] = outs[-2], outs[-1]

    kinds = {0: "h d", 1: "h u", 2: "u"}
    h3, hd3, hu3, s13, s23 = {}, {}, {}, {}, {}
    for i in (0, 1, 2):
        toks = kinds[i].split()
        ws = [l3[wmap[t]] for t in toks]
        sc = "h" in toks
        outs = _attn(laps[i], h2[i], s12[i], s22[i], 0,
                     bnds[i] if i > 0 else None, hd2.get(i - 1), 0,
                     bnds[i + 1], hu2.get(i + 1), 0,
                     ws, a3p[0] if sc else None, a3p[1] if sc else None)
        dsts = {"h": h3, "d": hd3, "u": hu3}
        for t, o in zip(toks, outs[:len(toks)]):
            dsts[t][i] = o
        if sc:
            s13[i], s23[i] = outs[-2], outs[-1]

    h40, s14, s24 = _attn(laps[0], h3[0], s13[0], s23[0], 0,
                          None, None, 0, bnds[1], hu3[1], 0,
                          [l4["W"]], a4p[0], a4p[1])
    (pu41,) = _attn(laps[1], h3[1], s13[1], s23[1], 0,
                    bnds[1], hd3[0], 0, bnds[2], hu3[2], 0,
                    [l4["Wu"]])

    rows = _final(Lg, idx.reshape(-1, 1), s14, s24, h40, Bg, pu41,
                  params["W_rel"], params["b_rel"].reshape(1, -1))

    nz = jnp.stack(jnp.nonzero(rel, size=rel.shape[0]), axis=1)
    return rows[nz]
